# baseline (speedup 1.0000x reference)
import dataclasses
import functools

import jax
import jax.numpy as jnp
from jax import lax
from jax.experimental import pallas as pl
from jax.experimental.pallas import tpu as pltpu
from jax.experimental.pallas import tpu_sc as plsc

B, L, F = 1024, 20, 26
NUM_FEAT, VOCAB, E = 128, 100000, 32
N = B * L

NC, NS = 2, 16
NW = NC * NS
N_PER_W = N // NW
GP = 4
G_IDX = GP * F
NG = N_PER_W // GP
LANES = 4 * E

VB = 1024
NB = 98
TROWS = VB // 4
TLINES = F * NB * TROWS


def _sc_compiler_params():
    cp = pltpu.CompilerParams(use_tc_tiling_on_sc=False)
    if "needs_layout_passes" in pltpu.CompilerParams.__dataclass_fields__:
        cp = dataclasses.replace(cp, needs_layout_passes=False)
    return cp


def _tp_body(x_ref, o_ref):
    x = x_ref[0]
    parts = [x[:, 256 * c:256 * (c + 1)].T for c in range(4)]
    o_ref[...] = jnp.concatenate(parts, axis=1)


def _transpose_table(tt):
    return pl.pallas_call(
        _tp_body,
        grid=(F, NB),
        in_specs=[pl.BlockSpec((1, E, VB), lambda f, j: (f, 0, j))],
        out_specs=pl.BlockSpec((TROWS, LANES), lambda f, j: (f * NB + j, 0)),
        out_shape=jax.ShapeDtypeStruct((TLINES, LANES), jnp.float32),
    )(tt)


def _mm_body(x_ref, w_ref, o_ref):
    o_ref[...] = lax.dot_general(
        x_ref[...], w_ref[...],
        (((1,), (1,)), ((), ())),
        preferred_element_type=jnp.float32,
    )


def _matmul(x, w):
    blk = 2048
    return pl.pallas_call(
        _mm_body,
        grid=(N // blk,),
        in_specs=[
            pl.BlockSpec((blk, NUM_FEAT), lambda i: (i, 0)),
            pl.BlockSpec((E, NUM_FEAT), lambda i: (0, 0)),
        ],
        out_specs=pl.BlockSpec((blk, E), lambda i: (i, 0)),
        out_shape=jax.ShapeDtypeStruct((N, E), jnp.float32),
    )(x, w)


def _sc_body(table_hbm, lines_hbm, sub_hbm, e2_hbm, out_hbm,
             lines_v, sub_v, acc_v, rows0, rows1, sem0, sem1):
    cid = lax.axis_index("core")
    sid = lax.axis_index("subcore")
    wid = sid * NC + cid
    base = wid * (N_PER_W * E)

    pltpu.sync_copy(lines_hbm.at[wid], lines_v)
    pltpu.sync_copy(sub_hbm.at[wid], sub_v)
    pltpu.sync_copy(e2_hbm.at[pl.ds(base, N_PER_W * E)], acc_v)

    iota16 = lax.iota(jnp.int32, 16)

    def fire(g, buf, sem):
        pltpu.async_copy(table_hbm.at[lines_v.at[g]], buf, sem)

    def wait(g, buf, sem):
        pltpu.make_async_copy(table_hbm.at[lines_v.at[g]], buf, sem).wait()

    def accum(g, buf):
        gvec = jnp.zeros((16,), jnp.int32) + g
        for q in range(GP):
            off = (g * GP + q) * E
            a0 = acc_v[pl.ds(off, 16)]
            a1 = acc_v[pl.ds(off + 16, 16)]
            for f in range(F):
                r = q * F + f
                rvec = jnp.full((16,), r, jnp.int32)
                col0 = plsc.load_gather(sub_v, [gvec, rvec]) + iota16
                a0 = a0 + plsc.load_gather(buf, [rvec, col0])
                a1 = a1 + plsc.load_gather(buf, [rvec, col0 + 16])
            acc_v[pl.ds(off, 16)] = a0
            acc_v[pl.ds(off + 16, 16)] = a1

    fire(0, rows0, sem0)
    fire(1, rows1, sem1)

    @pl.loop(0, NG, step=2)
    def _(g):
        wait(g, rows0, sem0)
        accum(g, rows0)

        @pl.when(g + 2 < NG)
        def _():
            fire(g + 2, rows0, sem0)

        wait(g + 1, rows1, sem1)
        accum(g + 1, rows1)

        @pl.when(g + 3 < NG)
        def _():
            fire(g + 3, rows1, sem1)

    pltpu.sync_copy(acc_v, out_hbm.at[pl.ds(base, N_PER_W * E)])


@jax.jit
def kernel(nodes_numerical, nodes_categorical, W_num, tables):
    x = nodes_numerical.reshape(N, NUM_FEAT)
    e2 = _matmul(x, W_num).reshape(N * E)

    table_lines = _transpose_table(tables.transpose(0, 2, 1))

    v = nodes_categorical.reshape(N, F)
    fbase = (jnp.arange(F, dtype=jnp.int32) * (NB * TROWS))[None, :]
    lines = (fbase + ((v >> 10) << 8) + (v & 255)).reshape(NW, NG, G_IDX)
    sub = (((v >> 8) & 3) * E).reshape(NW, NG, G_IDX)

    mesh = plsc.VectorSubcoreMesh(core_axis_name="core",
                                  subcore_axis_name="subcore")
    sc = pl.kernel(
        _sc_body,
        out_type=jax.ShapeDtypeStruct((N * E,), jnp.float32),
        mesh=mesh,
        scratch_types=[
            pltpu.VMEM((NG, G_IDX), jnp.int32),
            pltpu.VMEM((NG, G_IDX), jnp.int32),
            pltpu.VMEM((N_PER_W * E,), jnp.float32),
            pltpu.VMEM((G_IDX, LANES), jnp.float32),
            pltpu.VMEM((G_IDX, LANES), jnp.float32),
            pltpu.SemaphoreType.DMA,
            pltpu.SemaphoreType.DMA,
        ],
        compiler_params=_sc_compiler_params(),
    )
    out = sc(table_lines, lines, sub, e2)
    return out.reshape(B, L, E)


# speedup vs baseline: 2.5462x; 2.5462x over previous
import dataclasses
import functools

import jax
import jax.numpy as jnp
from jax import lax
from jax.experimental import pallas as pl
from jax.experimental.pallas import tpu as pltpu
from jax.experimental.pallas import tpu_sc as plsc

B, L, F = 1024, 20, 26
NUM_FEAT, VOCAB, E = 128, 100000, 32
N = B * L

NC, NS = 2, 16
NW = NC * NS
N_PER_W = N // NW
GP = 4
G_IDX = GP * F
NG = N_PER_W // GP
LANES = 4 * E

VB = 1024
NB = 98
NQ = 7
QLINES = NB * VB
TLINES = NQ * QLINES


def _sc_compiler_params():
    cp = pltpu.CompilerParams(use_tc_tiling_on_sc=False)
    if "needs_layout_passes" in pltpu.CompilerParams.__dataclass_fields__:
        cp = dataclasses.replace(cp, needs_layout_passes=False)
    return cp


def _tp_body(x_ref, o_ref):
    x4 = x_ref[...].reshape(4 * E, VB)
    o_ref[...] = x4.T


def _transpose_table(tt):
    return pl.pallas_call(
        _tp_body,
        grid=(NQ, NB),
        in_specs=[pl.BlockSpec((4, E, VB), lambda q, j: (q, 0, j))],
        out_specs=pl.BlockSpec((VB, LANES), lambda q, j: (q * NB + j, 0)),
        out_shape=jax.ShapeDtypeStruct((TLINES, LANES), jnp.float32),
    )(tt)


def _mm_body(x_ref, w_ref, o_ref):
    o_ref[...] = lax.dot_general(
        x_ref[...], w_ref[...],
        (((1,), (1,)), ((), ())),
        preferred_element_type=jnp.float32,
    )


def _matmul(x, w):
    blk = 2048
    return pl.pallas_call(
        _mm_body,
        grid=(N // blk,),
        in_specs=[
            pl.BlockSpec((blk, NUM_FEAT), lambda i: (i, 0)),
            pl.BlockSpec((E, NUM_FEAT), lambda i: (0, 0)),
        ],
        out_specs=pl.BlockSpec((blk, E), lambda i: (i, 0)),
        out_shape=jax.ShapeDtypeStruct((N, E), jnp.float32),
    )(x, w)


def _sc_body(table_hbm, lines_hbm, e2_hbm, out_hbm,
             lines_v, acc_v, rows0, rows1, sem0, sem1):
    cid = lax.axis_index("core")
    sid = lax.axis_index("subcore")
    wid = sid * NC + cid
    base = wid * (N_PER_W * E)

    pltpu.sync_copy(lines_hbm.at[wid], lines_v)
    pltpu.sync_copy(e2_hbm.at[pl.ds(base, N_PER_W * E)], acc_v)

    iota16 = lax.iota(jnp.int32, 16)

    def fire(g, buf, sem):
        pltpu.async_copy(table_hbm.at[lines_v.at[g]], buf, sem)

    def wait(g, buf, sem):
        pltpu.make_async_copy(table_hbm.at[lines_v.at[g]], buf, sem).wait()

    def accum(g, buf):
        for q in range(GP):
            off = (g * GP + q) * E
            a0 = acc_v[pl.ds(off, 16)]
            a1 = acc_v[pl.ds(off + 16, 16)]
            for f in range(F):
                r = q * F + f
                rvec = jnp.full((16,), r, jnp.int32)
                col0 = iota16 + ((f & 3) * E)
                a0 = a0 + plsc.load_gather(buf, [rvec, col0])
                a1 = a1 + plsc.load_gather(buf, [rvec, col0 + 16])
            acc_v[pl.ds(off, 16)] = a0
            acc_v[pl.ds(off + 16, 16)] = a1

    fire(0, rows0, sem0)
    fire(1, rows1, sem1)

    @pl.loop(0, NG, step=2)
    def _(g):
        wait(g, rows0, sem0)
        accum(g, rows0)

        @pl.when(g + 2 < NG)
        def _():
            fire(g + 2, rows0, sem0)

        wait(g + 1, rows1, sem1)
        accum(g + 1, rows1)

        @pl.when(g + 3 < NG)
        def _():
            fire(g + 3, rows1, sem1)

    pltpu.sync_copy(acc_v, out_hbm.at[pl.ds(base, N_PER_W * E)])


@jax.jit
def kernel(nodes_numerical, nodes_categorical, W_num, tables):
    x = nodes_numerical.reshape(N, NUM_FEAT)
    e2 = _matmul(x, W_num).reshape(N * E)

    table_lines = _transpose_table(tables.transpose(0, 2, 1))

    v = nodes_categorical.reshape(N, F)
    fbase = ((jnp.arange(F, dtype=jnp.int32) >> 2) * QLINES)[None, :]
    lines = (v + fbase).reshape(NW, NG, G_IDX)

    mesh = plsc.VectorSubcoreMesh(core_axis_name="core",
                                  subcore_axis_name="subcore")
    sc = pl.kernel(
        _sc_body,
        out_type=jax.ShapeDtypeStruct((N * E,), jnp.float32),
        mesh=mesh,
        scratch_types=[
            pltpu.VMEM((NG, G_IDX), jnp.int32),
            pltpu.VMEM((N_PER_W * E,), jnp.float32),
            pltpu.VMEM((G_IDX, LANES), jnp.float32),
            pltpu.VMEM((G_IDX, LANES), jnp.float32),
            pltpu.SemaphoreType.DMA,
            pltpu.SemaphoreType.DMA,
        ],
        compiler_params=_sc_compiler_params(),
    )
    out = sc(table_lines, lines, e2)
    return out.reshape(B, L, E)


# speedup vs baseline: 3.7961x; 1.4909x over previous
import dataclasses
import functools

import jax
import jax.numpy as jnp
from jax import lax
from jax.experimental import pallas as pl
from jax.experimental.pallas import tpu as pltpu
from jax.experimental.pallas import tpu_sc as plsc

B, L, F = 1024, 20, 26
NUM_FEAT, VOCAB, E = 128, 100000, 32
N = B * L

NC, NS = 2, 16
NW = NC * NS
N_PER_W = N // NW
GP = 4
G_IDX = GP * F
NG = N_PER_W // GP
LANES = 4 * E

VB = 4096
NB = 25
NQ = 7
QLINES = NB * VB
TLINES = NQ * QLINES


def _sc_compiler_params():
    cp = pltpu.CompilerParams(use_tc_tiling_on_sc=False)
    if "needs_layout_passes" in pltpu.CompilerParams.__dataclass_fields__:
        cp = dataclasses.replace(cp, needs_layout_passes=False)
    return cp


def _tp_body(x_ref, o_ref):
    x4 = x_ref[...].reshape(4 * E, VB)
    o_ref[...] = x4.T


def _transpose_table(tt):
    return pl.pallas_call(
        _tp_body,
        grid=(NQ, NB),
        in_specs=[pl.BlockSpec((4, E, VB), lambda q, j: (q, 0, j))],
        out_specs=pl.BlockSpec((VB, LANES), lambda q, j: (q * NB + j, 0)),
        out_shape=jax.ShapeDtypeStruct((TLINES, LANES), jnp.float32),
    )(tt)


def _mm_body(x_ref, w_ref, o_ref):
    o_ref[...] = lax.dot_general(
        x_ref[...], w_ref[...],
        (((1,), (1,)), ((), ())),
        preferred_element_type=jnp.float32,
    )


def _matmul(x, w):
    blk = 2048
    return pl.pallas_call(
        _mm_body,
        grid=(N // blk,),
        in_specs=[
            pl.BlockSpec((blk, NUM_FEAT), lambda i: (i, 0)),
            pl.BlockSpec((E, NUM_FEAT), lambda i: (0, 0)),
        ],
        out_specs=pl.BlockSpec((blk, E), lambda i: (i, 0)),
        out_shape=jax.ShapeDtypeStruct((N, E), jnp.float32),
    )(x, w)


def _sc_body(table_hbm, lines_hbm, e2_hbm, out_hbm,
             lines_v, acc_v, rows0, rows1, sem0, sem1):
    cid = lax.axis_index("core")
    sid = lax.axis_index("subcore")
    wid = sid * NC + cid
    base = wid * (N_PER_W * E)

    pltpu.sync_copy(lines_hbm.at[wid], lines_v)
    pltpu.sync_copy(e2_hbm.at[pl.ds(base, N_PER_W * E)], acc_v)

    iota16 = lax.iota(jnp.int32, 16)

    def fire(g, buf, sem):
        pltpu.async_copy(table_hbm.at[lines_v.at[g]], buf, sem)

    def wait(g, buf, sem):
        pltpu.make_async_copy(table_hbm.at[lines_v.at[g]], buf, sem).wait()

    def accum(g, buf):
        for q in range(GP):
            off = (g * GP + q) * E
            a0 = acc_v[pl.ds(off, 16)]
            a1 = acc_v[pl.ds(off + 16, 16)]
            for f in range(F):
                r = q * F + f
                rvec = jnp.full((16,), r, jnp.int32)
                col0 = iota16 + ((f & 3) * E)
                a0 = a0 + plsc.load_gather(buf, [rvec, col0])
                a1 = a1 + plsc.load_gather(buf, [rvec, col0 + 16])
            acc_v[pl.ds(off, 16)] = a0
            acc_v[pl.ds(off + 16, 16)] = a1

    fire(0, rows0, sem0)
    fire(1, rows1, sem1)

    @pl.loop(0, NG, step=2)
    def _(g):
        wait(g, rows0, sem0)
        accum(g, rows0)

        @pl.when(g + 2 < NG)
        def _():
            fire(g + 2, rows0, sem0)

        wait(g + 1, rows1, sem1)
        accum(g + 1, rows1)

        @pl.when(g + 3 < NG)
        def _():
            fire(g + 3, rows1, sem1)

    pltpu.sync_copy(acc_v, out_hbm.at[pl.ds(base, N_PER_W * E)])


@jax.jit
def kernel(nodes_numerical, nodes_categorical, W_num, tables):
    x = nodes_numerical.reshape(N, NUM_FEAT)
    e2 = _matmul(x, W_num).reshape(N * E)

    table_lines = _transpose_table(tables.transpose(0, 2, 1))

    v = nodes_categorical.reshape(N, F)
    fbase = ((jnp.arange(F, dtype=jnp.int32) >> 2) * QLINES)[None, :]
    lines = (v + fbase).reshape(NW, NG, G_IDX)

    mesh = plsc.VectorSubcoreMesh(core_axis_name="core",
                                  subcore_axis_name="subcore")
    sc = pl.kernel(
        _sc_body,
        out_type=jax.ShapeDtypeStruct((N * E,), jnp.float32),
        mesh=mesh,
        scratch_types=[
            pltpu.VMEM((NG, G_IDX), jnp.int32),
            pltpu.VMEM((N_PER_W * E,), jnp.float32),
            pltpu.VMEM((G_IDX, LANES), jnp.float32),
            pltpu.VMEM((G_IDX, LANES), jnp.float32),
            pltpu.SemaphoreType.DMA,
            pltpu.SemaphoreType.DMA,
        ],
        compiler_params=_sc_compiler_params(),
    )
    out = sc(table_lines, lines, e2)
    return out.reshape(B, L, E)


# speedup vs baseline: 4.0760x; 1.0737x over previous
import dataclasses
import functools

import jax
import jax.numpy as jnp
from jax import lax
from jax.experimental import pallas as pl
from jax.experimental.pallas import tpu as pltpu
from jax.experimental.pallas import tpu_sc as plsc

B, L, F = 1024, 20, 26
NUM_FEAT, VOCAB, E = 128, 100000, 32
N = B * L

NC, NS = 2, 16
NW = NC * NS
N_PER_W = N // NW
GP = 4
G_IDX = GP * F
NG = N_PER_W // GP
LANES = 4 * E

VB = 8192
NB = 13
NQ = 7
QLINES = NB * VB
TLINES = NQ * QLINES


def _sc_compiler_params():
    cp = pltpu.CompilerParams(use_tc_tiling_on_sc=False)
    if "needs_layout_passes" in pltpu.CompilerParams.__dataclass_fields__:
        cp = dataclasses.replace(cp, needs_layout_passes=False)
    return cp


def _tp_body(x_ref, o_ref):
    x4 = x_ref[...].reshape(4 * E, VB)
    o_ref[...] = x4.T


def _transpose_table(tt):
    return pl.pallas_call(
        _tp_body,
        grid=(NQ, NB),
        in_specs=[pl.BlockSpec((4, E, VB), lambda q, j: (q, 0, j))],
        out_specs=pl.BlockSpec((VB, LANES), lambda q, j: (q * NB + j, 0)),
        out_shape=jax.ShapeDtypeStruct((TLINES, LANES), jnp.float32),
    )(tt)


def _mm_body(x_ref, w_ref, o_ref):
    o_ref[...] = lax.dot_general(
        x_ref[...], w_ref[...],
        (((1,), (1,)), ((), ())),
        preferred_element_type=jnp.float32,
    )


def _matmul(x, w):
    blk = 2048
    return pl.pallas_call(
        _mm_body,
        grid=(N // blk,),
        in_specs=[
            pl.BlockSpec((blk, NUM_FEAT), lambda i: (i, 0)),
            pl.BlockSpec((E, NUM_FEAT), lambda i: (0, 0)),
        ],
        out_specs=pl.BlockSpec((blk, E), lambda i: (i, 0)),
        out_shape=jax.ShapeDtypeStruct((N, E), jnp.float32),
    )(x, w)


def _sc_body(table_hbm, lines_hbm, e2_hbm, out_hbm,
             lines_v, acc_v, rows0, rows1, sem0, sem1):
    cid = lax.axis_index("core")
    sid = lax.axis_index("subcore")
    wid = sid * NC + cid
    base = wid * (N_PER_W * E)

    pltpu.sync_copy(lines_hbm.at[wid], lines_v)
    pltpu.sync_copy(e2_hbm.at[pl.ds(base, N_PER_W * E)], acc_v)

    iota16 = lax.iota(jnp.int32, 16)

    def fire(g, buf, sem):
        pltpu.async_copy(table_hbm.at[lines_v.at[g]], buf, sem)

    def wait(g, buf, sem):
        pltpu.make_async_copy(table_hbm.at[lines_v.at[g]], buf, sem).wait()

    def accum(g, buf):
        for q in range(GP):
            off = (g * GP + q) * E
            a0 = acc_v[pl.ds(off, 16)]
            a1 = acc_v[pl.ds(off + 16, 16)]
            for f in range(F):
                r = q * F + f
                rvec = jnp.full((16,), r, jnp.int32)
                col0 = iota16 + ((f & 3) * E)
                a0 = a0 + plsc.load_gather(buf, [rvec, col0])
                a1 = a1 + plsc.load_gather(buf, [rvec, col0 + 16])
            acc_v[pl.ds(off, 16)] = a0
            acc_v[pl.ds(off + 16, 16)] = a1

    fire(0, rows0, sem0)
    fire(1, rows1, sem1)

    @pl.loop(0, NG, step=2)
    def _(g):
        wait(g, rows0, sem0)
        accum(g, rows0)

        @pl.when(g + 2 < NG)
        def _():
            fire(g + 2, rows0, sem0)

        wait(g + 1, rows1, sem1)
        accum(g + 1, rows1)

        @pl.when(g + 3 < NG)
        def _():
            fire(g + 3, rows1, sem1)

    pltpu.sync_copy(acc_v, out_hbm.at[pl.ds(base, N_PER_W * E)])


@jax.jit
def kernel(nodes_numerical, nodes_categorical, W_num, tables):
    x = nodes_numerical.reshape(N, NUM_FEAT)
    e2 = _matmul(x, W_num).reshape(N * E)

    table_lines = _transpose_table(tables.transpose(0, 2, 1))

    v = nodes_categorical.reshape(N, F)
    fbase = ((jnp.arange(F, dtype=jnp.int32) >> 2) * QLINES)[None, :]
    lines = (v + fbase).reshape(NW, NG, G_IDX)

    mesh = plsc.VectorSubcoreMesh(core_axis_name="core",
                                  subcore_axis_name="subcore")
    sc = pl.kernel(
        _sc_body,
        out_type=jax.ShapeDtypeStruct((N * E,), jnp.float32),
        mesh=mesh,
        scratch_types=[
            pltpu.VMEM((NG, G_IDX), jnp.int32),
            pltpu.VMEM((N_PER_W * E,), jnp.float32),
            pltpu.VMEM((G_IDX, LANES), jnp.float32),
            pltpu.VMEM((G_IDX, LANES), jnp.float32),
            pltpu.SemaphoreType.DMA,
            pltpu.SemaphoreType.DMA,
        ],
        compiler_params=_sc_compiler_params(),
    )
    out = sc(table_lines, lines, e2)
    return out.reshape(B, L, E)
